# Initial kernel scaffold; baseline (speedup 1.0000x reference)
import jax
import jax.numpy as jnp
from jax.experimental import pallas as pl


def kernel(tokens, W, b):
    raise NotImplementedError("write your pallas kernel here")


# speedup vs baseline: 6.7935x; 6.7935x over previous
import functools

import jax
import jax.numpy as jnp
from jax import lax
from jax.experimental import pallas as pl
from jax.experimental.pallas import tpu as pltpu
from jax.experimental.pallas import tpu_sc as plsc

_VOCAB = 100000
_OUT = 64
_B = 1024
_L = 200

_NC = 2
_NS = 16
_NW = _NC * _NS
_RPW = _B // _NW
_LANES = 16
_CHUNKS = _OUT // _LANES


def _bow_body(tok_hbm, w_hbm, b_hbm, out_hbm, idx_v, rows_v, b_v, out_v, sem):
    wid = lax.axis_index("s") * _NC + lax.axis_index("c")
    base = wid * _RPW

    pltpu.sync_copy(tok_hbm.at[pl.ds(base * _L, _RPW * _L)], idx_v)
    pltpu.sync_copy(b_hbm, b_v)

    bias = [b_v[pl.ds(c * _LANES, _LANES)] for c in range(_CHUNKS)]

    @pl.loop(0, _RPW)
    def _(r):
        pltpu.async_copy(
            w_hbm.at[idx_v.at[pl.ds(r * _L, _L)]], rows_v, sem
        ).wait()

        def acc_body(j, accs):
            return tuple(
                accs[c] + rows_v[j, pl.ds(c * _LANES, _LANES)]
                for c in range(_CHUNKS)
            )

        zero = jnp.zeros((_LANES,), jnp.float32)
        accs = lax.fori_loop(0, _L, acc_body, (zero,) * _CHUNKS)

        for c in range(_CHUNKS):
            out_v[r, pl.ds(c * _LANES, _LANES)] = jnp.maximum(
                accs[c] + bias[c], 0.0
            )

    pltpu.sync_copy(out_v, out_hbm.at[pl.ds(base, _RPW)])


@jax.jit
def kernel(tokens, W, b):
    tok = tokens.reshape(-1).astype(jnp.int32)
    run = functools.partial(
        pl.kernel,
        out_type=jax.ShapeDtypeStruct((_B, _OUT), jnp.float32),
        mesh=plsc.VectorSubcoreMesh(core_axis_name="c", subcore_axis_name="s"),
        scratch_types=[
            pltpu.VMEM((_RPW * _L,), jnp.int32),
            pltpu.VMEM((_L, _OUT), jnp.float32),
            pltpu.VMEM((_OUT,), jnp.float32),
            pltpu.VMEM((_RPW, _OUT), jnp.float32),
            pltpu.SemaphoreType.DMA,
        ],
        compiler_params=pltpu.CompilerParams(use_tc_tiling_on_sc=False),
    )(_bow_body)
    return run(tok, W, b)


# speedup vs baseline: 8.4328x; 1.2413x over previous
import functools

import jax
import jax.numpy as jnp
from jax import lax
from jax.experimental import pallas as pl
from jax.experimental.pallas import tpu as pltpu
from jax.experimental.pallas import tpu_sc as plsc

_VOCAB = 100000
_OUT = 64
_B = 1024
_L = 200

_NC = 2
_NS = 16
_NW = _NC * _NS
_RPW = _B // _NW
_LANES = 16
_CHUNKS = _OUT // _LANES


def _bow_body(
    tok_hbm, w_hbm, b_hbm, out_hbm, idx_v, rows0, rows1, b_v, out_v, sem0, sem1
):
    wid = lax.axis_index("s") * _NC + lax.axis_index("c")
    base = wid * _RPW

    pltpu.sync_copy(tok_hbm.at[pl.ds(base * _L, _RPW * _L)], idx_v)
    pltpu.sync_copy(b_hbm, b_v)

    bias = [b_v[pl.ds(c * _LANES, _LANES)] for c in range(_CHUNKS)]

    def gather(r, buf, sem):
        return pltpu.make_async_copy(
            w_hbm.at[idx_v.at[pl.ds(r * _L, _L)]], buf, sem
        )

    gather(0, rows0, sem0).start()
    gather(1, rows1, sem1).start()

    @pl.loop(0, _RPW, step=2)
    def _(r):
        for k, (buf, sem) in enumerate(((rows0, sem0), (rows1, sem1))):
            rr = r + k
            gather(rr, buf, sem).wait()

            def acc_body(j, accs, buf=buf):
                return tuple(
                    accs[c] + buf[j, pl.ds(c * _LANES, _LANES)]
                    for c in range(_CHUNKS)
                )

            zero = jnp.zeros((_LANES,), jnp.float32)
            accs = lax.fori_loop(0, _L, acc_body, (zero,) * _CHUNKS)

            @pl.when(rr + 2 < _RPW)
            def _(buf=buf, sem=sem, rr=rr):
                gather(rr + 2, buf, sem).start()

            for c in range(_CHUNKS):
                out_v[rr, pl.ds(c * _LANES, _LANES)] = jnp.maximum(
                    accs[c] + bias[c], 0.0
                )

    pltpu.sync_copy(out_v, out_hbm.at[pl.ds(base, _RPW)])


@jax.jit
def kernel(tokens, W, b):
    tok = tokens.reshape(-1).astype(jnp.int32)
    run = functools.partial(
        pl.kernel,
        out_type=jax.ShapeDtypeStruct((_B, _OUT), jnp.float32),
        mesh=plsc.VectorSubcoreMesh(core_axis_name="c", subcore_axis_name="s"),
        scratch_types=[
            pltpu.VMEM((_RPW * _L,), jnp.int32),
            pltpu.VMEM((_L, _OUT), jnp.float32),
            pltpu.VMEM((_L, _OUT), jnp.float32),
            pltpu.VMEM((_OUT,), jnp.float32),
            pltpu.VMEM((_RPW, _OUT), jnp.float32),
            pltpu.SemaphoreType.DMA,
            pltpu.SemaphoreType.DMA,
        ],
        compiler_params=pltpu.CompilerParams(use_tc_tiling_on_sc=False),
    )(_bow_body)
    return run(tok, W, b)


# speedup vs baseline: 8.6877x; 1.0302x over previous
import functools

import jax
import jax.numpy as jnp
from jax import lax
from jax.experimental import pallas as pl
from jax.experimental.pallas import tpu as pltpu
from jax.experimental.pallas import tpu_sc as plsc

_VOCAB = 100000
_OUT = 64
_B = 1024
_L = 200

_NC = 2
_NS = 16
_NW = _NC * _NS
_RPW = _B // _NW
_LANES = 16
_CHUNKS = _OUT // _LANES
_UNROLL = 8


def _bow_body(
    tok_hbm, w_hbm, b_hbm, out_hbm, idx_v, rows0, rows1, b_v, out_v, sem0, sem1
):
    wid = lax.axis_index("s") * _NC + lax.axis_index("c")
    base = wid * _RPW

    pltpu.sync_copy(tok_hbm.at[pl.ds(base * _L, _RPW * _L)], idx_v)
    pltpu.sync_copy(b_hbm, b_v)

    bias = [b_v[pl.ds(c * _LANES, _LANES)] for c in range(_CHUNKS)]

    def gather(r, buf, sem):
        return pltpu.make_async_copy(
            w_hbm.at[idx_v.at[pl.ds(r * _L, _L)]], buf, sem
        )

    gather(0, rows0, sem0).start()
    gather(1, rows1, sem1).start()

    @pl.loop(0, _RPW, step=2)
    def _(r):
        for k, (buf, sem) in enumerate(((rows0, sem0), (rows1, sem1))):
            rr = r + k
            gather(rr, buf, sem).wait()

            def acc_body(j, accs, buf=buf):
                accs = list(accs)
                row = j * _UNROLL
                for u in range(_UNROLL):
                    for c in range(_CHUNKS):
                        a = (u % 2) * _CHUNKS + c
                        accs[a] = accs[a] + buf[row + u, pl.ds(c * _LANES, _LANES)]
                return tuple(accs)

            zero = jnp.zeros((_LANES,), jnp.float32)
            accs = lax.fori_loop(0, _L // _UNROLL, acc_body, (zero,) * (2 * _CHUNKS))
            accs = [accs[c] + accs[_CHUNKS + c] for c in range(_CHUNKS)]

            @pl.when(rr + 2 < _RPW)
            def _(buf=buf, sem=sem, rr=rr):
                gather(rr + 2, buf, sem).start()

            for c in range(_CHUNKS):
                out_v[rr, pl.ds(c * _LANES, _LANES)] = jnp.maximum(
                    accs[c] + bias[c], 0.0
                )

    pltpu.sync_copy(out_v, out_hbm.at[pl.ds(base, _RPW)])


@jax.jit
def kernel(tokens, W, b):
    tok = tokens.reshape(-1).astype(jnp.int32)
    run = functools.partial(
        pl.kernel,
        out_type=jax.ShapeDtypeStruct((_B, _OUT), jnp.float32),
        mesh=plsc.VectorSubcoreMesh(core_axis_name="c", subcore_axis_name="s"),
        scratch_types=[
            pltpu.VMEM((_RPW * _L,), jnp.int32),
            pltpu.VMEM((_L, _OUT), jnp.float32),
            pltpu.VMEM((_L, _OUT), jnp.float32),
            pltpu.VMEM((_OUT,), jnp.float32),
            pltpu.VMEM((_RPW, _OUT), jnp.float32),
            pltpu.SemaphoreType.DMA,
            pltpu.SemaphoreType.DMA,
        ],
        compiler_params=pltpu.CompilerParams(use_tc_tiling_on_sc=False),
    )(_bow_body)
    return run(tok, W, b)
